# baseline (speedup 1.0000x reference)
import jax
import jax.numpy as jnp
from jax.experimental import pallas as pl
from jax.experimental.pallas import tpu as pltpu

_NS = 2
_NT = 8


def _gcn_kernel(x_ref, adj_ref, w_ref, b_ref, o_ref, acc_ref, mask_ref):
    t = pl.program_id(1)
    tK = x_ref.shape[0]
    B = mask_ref.shape[1] // w_ref.shape[1]

    @pl.when(t == 0)
    def _():
        acc_ref[...] = jnp.zeros_like(acc_ref)
        row = jax.lax.broadcasted_iota(jnp.int32, mask_ref.shape, 0)
        col = jax.lax.broadcasted_iota(jnp.int32, mask_ref.shape, 1)
        mask_ref[...] = (col // w_ref.shape[1] == row % B).astype(jnp.float32)

    y = jnp.dot(x_ref[...], w_ref[...],
                preferred_element_type=jnp.float32) + b_ref[...]
    ybd = jnp.concatenate([y] * B, axis=1) * mask_ref[...]
    acc_ref[...] += jnp.dot(adj_ref[...], ybd,
                            preferred_element_type=jnp.float32)

    @pl.when(t == pl.num_programs(1) - 1)
    def _():
        o_ref[...] = jnp.maximum(acc_ref[...], 0.0)


def kernel(x, adj, w, b):
    S, B, H = x.shape
    O = w.shape[0]
    tS = S // _NS
    tK = (S * B) // _NT

    x2 = x.reshape(S * B, H)
    adj2 = adj.reshape(S, S * B)
    w_t = jnp.transpose(w)
    b2d = b.reshape(1, O).astype(jnp.float32)

    out2 = pl.pallas_call(
        _gcn_kernel,
        out_shape=jax.ShapeDtypeStruct((S, B * O), jnp.float32),
        grid_spec=pltpu.PrefetchScalarGridSpec(
            num_scalar_prefetch=0,
            grid=(_NS, _NT),
            in_specs=[
                pl.BlockSpec((tK, H), lambda i, t: (t, 0)),
                pl.BlockSpec((tS, tK), lambda i, t: (i, t)),
                pl.BlockSpec((H, O), lambda i, t: (0, 0)),
                pl.BlockSpec((1, O), lambda i, t: (0, 0)),
            ],
            out_specs=pl.BlockSpec((tS, B * O), lambda i, t: (i, 0)),
            scratch_shapes=[
                pltpu.VMEM((tS, B * O), jnp.float32),
                pltpu.VMEM((tK, B * O), jnp.float32),
            ],
        ),
        compiler_params=pltpu.CompilerParams(
            dimension_semantics=("parallel", "arbitrary"),
            vmem_limit_bytes=100 * 1024 * 1024,
        ),
    )(x2, adj2, w_t, b2d)

    return out2.reshape(S, B, O)


# speedup vs baseline: 2.0707x; 2.0707x over previous
import jax
import jax.numpy as jnp
from jax.experimental import pallas as pl
from jax.experimental.pallas import tpu as pltpu

_NS = 2


def _gcn_kernel(x_ref, adj_ref, w_ref, b_ref, o_ref):
    b = pl.program_id(1)
    x_b = x_ref[:, b, :]
    y = jnp.dot(x_b.astype(jnp.bfloat16), w_ref[...],
                preferred_element_type=jnp.float32) + b_ref[...]
    z = jnp.dot(adj_ref[...], y.astype(jnp.bfloat16),
                preferred_element_type=jnp.float32)
    o_ref[:, b, :] = jnp.maximum(z, 0.0)


def kernel(x, adj, w, b):
    S, B, H = x.shape
    O = w.shape[0]
    tS = S // _NS

    adj_bm = jnp.transpose(adj.astype(jnp.bfloat16), (2, 0, 1))
    w_t = jnp.transpose(w).astype(jnp.bfloat16)
    b2d = b.reshape(1, O).astype(jnp.float32)

    return pl.pallas_call(
        _gcn_kernel,
        out_shape=jax.ShapeDtypeStruct((S, B, O), jnp.float32),
        grid_spec=pltpu.PrefetchScalarGridSpec(
            num_scalar_prefetch=0,
            grid=(_NS, B),
            in_specs=[
                pl.BlockSpec((S, B, H), lambda i, j: (0, 0, 0)),
                pl.BlockSpec((None, tS, S), lambda i, j: (j, i, 0)),
                pl.BlockSpec((H, O), lambda i, j: (0, 0)),
                pl.BlockSpec((1, O), lambda i, j: (0, 0)),
            ],
            out_specs=pl.BlockSpec((tS, B, O), lambda i, j: (i, 0, 0)),
        ),
        compiler_params=pltpu.CompilerParams(
            dimension_semantics=("parallel", "arbitrary"),
            vmem_limit_bytes=64 * 1024 * 1024,
        ),
    )(x, adj_bm, w_t, b2d)


# speedup vs baseline: 2.1401x; 1.0335x over previous
import jax
import jax.numpy as jnp
from jax.experimental import pallas as pl
from jax.experimental.pallas import tpu as pltpu

_NS = 2


def _gcn_kernel(x_ref, adj_ref, w_ref, b_ref, o_ref, y_ref):
    S, B, H = x_ref.shape
    O = w_ref.shape[1]
    b = pl.program_id(1)

    @pl.when(b == 0)
    def _():
        xf = x_ref[...].reshape(S * B, H).astype(jnp.bfloat16)
        y = jnp.dot(xf, w_ref[...],
                    preferred_element_type=jnp.float32) + b_ref[...]
        y3 = y.astype(jnp.bfloat16).reshape(S, B, O)
        for bb in range(B):
            y_ref[bb * S:(bb + 1) * S, :] = y3[:, bb, :]

    y_b = y_ref[pl.ds(b * S, S), :]
    z = jnp.dot(adj_ref[...], y_b,
                preferred_element_type=jnp.float32)
    o_ref[:, b, :] = jnp.maximum(z, 0.0)


def kernel(x, adj, w, b):
    S, B, H = x.shape
    O = w.shape[0]
    tS = S // _NS

    adj_bm = jnp.transpose(adj.astype(jnp.bfloat16), (2, 0, 1))
    w_t = jnp.transpose(w).astype(jnp.bfloat16)
    b2d = b.reshape(1, O).astype(jnp.float32)

    return pl.pallas_call(
        _gcn_kernel,
        out_shape=jax.ShapeDtypeStruct((S, B, O), jnp.float32),
        grid_spec=pltpu.PrefetchScalarGridSpec(
            num_scalar_prefetch=0,
            grid=(_NS, B),
            in_specs=[
                pl.BlockSpec((S, B, H), lambda i, j: (0, 0, 0)),
                pl.BlockSpec((None, tS, S), lambda i, j: (j, i, 0)),
                pl.BlockSpec((H, O), lambda i, j: (0, 0)),
                pl.BlockSpec((1, O), lambda i, j: (0, 0)),
            ],
            out_specs=pl.BlockSpec((tS, B, O), lambda i, j: (i, 0, 0)),
            scratch_shapes=[pltpu.VMEM((B * S, O), jnp.bfloat16)],
        ),
        compiler_params=pltpu.CompilerParams(
            dimension_semantics=("parallel", "arbitrary"),
            vmem_limit_bytes=64 * 1024 * 1024,
        ),
    )(x, adj_bm, w_t, b2d)
